# baseline (speedup 1.0000x reference)
import jax
import jax.numpy as jnp
from jax import lax
from jax.experimental import pallas as pl
from jax.experimental.pallas import tpu as pltpu
from jax.experimental.pallas import tpu_sc as plsc

B = 1024
READOUT_DIM = 16384
TOTAL = B * READOUT_DIM
NC, NS, L = 2, 16, 16
NW = NC * NS
FR, FI = 4, 8
NF = FR * FI
PLANE = TOTAL // FR
SN = 512
STEP_ELEMS = SN * FI
W_NODES = 16384
STEPS = W_NODES // SN
ROW = READOUT_DIM
PITCH = 17
GROUPS = SN // L
NBUF = 2


def _sc_body(f_hbm, w_hbm, o_hbm, fv0, fv1, wv0, wv1, ov0, ov1, mb,
             semf, semw, semo):
    fvs = (fv0, fv1)
    wvs = (wv0, wv1)
    ovs = (ov0, ov1)
    wid = lax.axis_index("s") * NC + lax.axis_index("c")
    in_base = wid * (W_NODES * FI)
    row_base = wid * STEPS

    def start_in(s, b):
        for fr in range(FR):
            a = fr * PLANE + in_base + s * STEP_ELEMS
            d = pl.ds(fr * STEP_ELEMS, STEP_ELEMS)
            pltpu.make_async_copy(f_hbm.at[pl.ds(a, STEP_ELEMS)], fvs[b].at[d], semf.at[b]).start()
            pltpu.make_async_copy(w_hbm.at[pl.ds(a, STEP_ELEMS)], wvs[b].at[d], semw.at[b]).start()

    def wait_in(s, b):
        for fr in range(FR):
            a = fr * PLANE + in_base + s * STEP_ELEMS
            d = pl.ds(fr * STEP_ELEMS, STEP_ELEMS)
            pltpu.make_async_copy(f_hbm.at[pl.ds(a, STEP_ELEMS)], fvs[b].at[d], semf.at[b]).wait()
            pltpu.make_async_copy(w_hbm.at[pl.ds(a, STEP_ELEMS)], wvs[b].at[d], semw.at[b]).wait()

    def start_out(s, b):
        g = row_base + s
        pltpu.make_async_copy(ovs[b], o_hbm.at[g >> 3, :, g & 7, :], semo.at[b]).start()

    def wait_out(s, b):
        g = row_base + s
        pltpu.make_async_copy(ovs[b], o_hbm.at[g >> 3, :, g & 7, :], semo.at[b]).wait()

    lanes = lax.iota(jnp.int32, L)
    lanes17 = lanes * PITCH

    start_in(0, 0)

    @pl.loop(0, STEPS, step=NBUF)
    def outer(s0):
        for b in range(NBUF):
            s = s0 + b

            @pl.when(s + 1 < STEPS)
            def _():
                start_in(s + 1, 1 - b)

            wait_in(s, b)

            @pl.when(s >= NBUF)
            def _():
                wait_out(s - NBUF, b)

            ob = ovs[b]
            fb, wb = fvs[b], wvs[b]

            @pl.loop(0, GROUPS)
            def _grp(g16):
                in_off = (g16 >> 3) * 1024 + (g16 & 7) * L

                @plsc.parallel_loop(0, NF, unroll=8)
                def _p1(f):
                    off = (f >> 3) * STEP_ELEMS + (f & 7) * 128 + in_off
                    p = fb[pl.ds(off, L)] * wb[pl.ds(off, L)]
                    mb[pl.ds(f * PITCH, L)] = p

                ov_base = g16 * (L * NF)

                @plsc.parallel_loop(0, L, unroll=8, carry=lanes17)
                def _p2(j, colA):
                    a = plsc.load_gather(mb, [colA])
                    c = plsc.load_gather(mb, [colA + L * PITCH])
                    o = ov_base + j * NF
                    row = o >> 7
                    col = o & 96
                    ob[row, pl.ds(col, L)] = a
                    ob[row, pl.ds(col + L, L)] = c
                    return colA + 1

            start_out(s, b)

    wait_out(STEPS - 2, 0)
    wait_out(STEPS - 1, 1)


def kernel(features, weights):
    def phys_flat(x):
        return x.reshape(4096, 128, FR, FI).transpose(2, 0, 3, 1).reshape(TOTAL)

    f2 = phys_flat(features)
    w2 = phys_flat(weights)
    mesh = plsc.VectorSubcoreMesh(
        core_axis_name="c", subcore_axis_name="s",
        num_cores=NC, num_subcores=NS)
    out = pl.kernel(
        _sc_body,
        out_type=jax.ShapeDtypeStruct((128, 128, 8, 128), jnp.float32),
        mesh=mesh,
        compiler_params=pltpu.CompilerParams(needs_layout_passes=False),
        scratch_types=[
            pltpu.VMEM((FR * STEP_ELEMS,), jnp.float32),
            pltpu.VMEM((FR * STEP_ELEMS,), jnp.float32),
            pltpu.VMEM((FR * STEP_ELEMS,), jnp.float32),
            pltpu.VMEM((FR * STEP_ELEMS,), jnp.float32),
            pltpu.VMEM((128, 128), jnp.float32),
            pltpu.VMEM((128, 128), jnp.float32),
            pltpu.VMEM((NF * PITCH,), jnp.float32),
            pltpu.SemaphoreType.DMA((NBUF,)),
            pltpu.SemaphoreType.DMA((NBUF,)),
            pltpu.SemaphoreType.DMA((NBUF,)),
        ],
    )(f2, w2)
    return out.transpose(0, 2, 1, 3).reshape(B, READOUT_DIM)


# speedup vs baseline: 1.0061x; 1.0061x over previous
import jax
import jax.numpy as jnp
from jax import lax
from jax.experimental import pallas as pl
from jax.experimental.pallas import tpu as pltpu
from jax.experimental.pallas import tpu_sc as plsc

B = 1024
READOUT_DIM = 16384
TOTAL = B * READOUT_DIM
NC, NS, L = 2, 16, 16
NW = NC * NS
FR, FI = 4, 8
NF = FR * FI
PLANE = TOTAL // FR
SN = 512
STEP_ELEMS = SN * FI
W_NODES = 16384
STEPS = W_NODES // SN
ROW = READOUT_DIM
PITCH = 17
GROUPS = SN // L
NBUF = 2


def _sc_body(f_hbm, w_hbm, o_hbm, fv0, fv1, wv0, wv1, ov0, ov1, mb,
             semf, semw, semo):
    fvs = (fv0, fv1)
    wvs = (wv0, wv1)
    ovs = (ov0, ov1)
    wid = lax.axis_index("s") * NC + lax.axis_index("c")
    in_base = wid * (W_NODES * FI)
    row_base = wid * STEPS

    def start_in(s, b):
        for fr in range(FR):
            a = fr * PLANE + in_base + s * STEP_ELEMS
            d = pl.ds(fr * STEP_ELEMS, STEP_ELEMS)
            pltpu.make_async_copy(f_hbm.at[pl.ds(a, STEP_ELEMS)], fvs[b].at[d], semf.at[b]).start()
            pltpu.make_async_copy(w_hbm.at[pl.ds(a, STEP_ELEMS)], wvs[b].at[d], semw.at[b]).start()

    def wait_in(s, b):
        for fr in range(FR):
            a = fr * PLANE + in_base + s * STEP_ELEMS
            d = pl.ds(fr * STEP_ELEMS, STEP_ELEMS)
            pltpu.make_async_copy(f_hbm.at[pl.ds(a, STEP_ELEMS)], fvs[b].at[d], semf.at[b]).wait()
            pltpu.make_async_copy(w_hbm.at[pl.ds(a, STEP_ELEMS)], wvs[b].at[d], semw.at[b]).wait()

    def start_out(s, b):
        g = row_base + s
        pltpu.make_async_copy(ovs[b], o_hbm.at[g >> 3, :, g & 7, :], semo.at[b]).start()

    def wait_out(s, b):
        g = row_base + s
        pltpu.make_async_copy(ovs[b], o_hbm.at[g >> 3, :, g & 7, :], semo.at[b]).wait()

    lanes = lax.iota(jnp.int32, L)
    lanes17 = lanes * PITCH

    start_in(0, 0)

    @pl.loop(0, STEPS, step=NBUF)
    def outer(s0):
        for b in range(NBUF):
            s = s0 + b

            @pl.when(s + 1 < STEPS)
            def _():
                start_in(s + 1, 1 - b)

            wait_in(s, b)

            @pl.when(s >= NBUF)
            def _():
                wait_out(s - NBUF, b)

            ob = ovs[b]
            fb, wb = fvs[b], wvs[b]

            @pl.loop(0, GROUPS)
            def _grp(g16):
                in_off = (g16 >> 3) * 1024 + (g16 & 7) * L

                @plsc.parallel_loop(0, NF, unroll=4, carry=lanes)
                def _p1(f, mrow):
                    off = (f >> 3) * STEP_ELEMS + (f & 7) * 128 + in_off
                    p = fb[pl.ds(off, L)] * wb[pl.ds(off, L)]
                    plsc.store_scatter(mb, [mrow], p)
                    return mrow + PITCH

                ov_base = g16 * (L * NF)

                @plsc.parallel_loop(0, L, unroll=8, carry=lanes17)
                def _p2(j, colA):
                    a = plsc.load_gather(mb, [colA])
                    c = plsc.load_gather(mb, [colA + L * PITCH])
                    o = ov_base + j * NF
                    row = o >> 7
                    col = o & 96
                    ob[row, pl.ds(col, L)] = a
                    ob[row, pl.ds(col + L, L)] = c
                    return colA + 1

            start_out(s, b)

    wait_out(STEPS - 2, 0)
    wait_out(STEPS - 1, 1)


def kernel(features, weights):
    def phys_flat(x):
        return x.reshape(4096, 128, FR, FI).transpose(2, 0, 3, 1).reshape(TOTAL)

    f2 = phys_flat(features)
    w2 = phys_flat(weights)
    mesh = plsc.VectorSubcoreMesh(
        core_axis_name="c", subcore_axis_name="s",
        num_cores=NC, num_subcores=NS)
    out = pl.kernel(
        _sc_body,
        out_type=jax.ShapeDtypeStruct((128, 128, 8, 128), jnp.float32),
        mesh=mesh,
        compiler_params=pltpu.CompilerParams(needs_layout_passes=False),
        scratch_types=[
            pltpu.VMEM((FR * STEP_ELEMS,), jnp.float32),
            pltpu.VMEM((FR * STEP_ELEMS,), jnp.float32),
            pltpu.VMEM((FR * STEP_ELEMS,), jnp.float32),
            pltpu.VMEM((FR * STEP_ELEMS,), jnp.float32),
            pltpu.VMEM((128, 128), jnp.float32),
            pltpu.VMEM((128, 128), jnp.float32),
            pltpu.VMEM((NF * PITCH,), jnp.float32),
            pltpu.SemaphoreType.DMA((NBUF,)),
            pltpu.SemaphoreType.DMA((NBUF,)),
            pltpu.SemaphoreType.DMA((NBUF,)),
        ],
    )(f2, w2)
    return out.transpose(0, 2, 1, 3).reshape(B, READOUT_DIM)


# speedup vs baseline: 1.0993x; 1.0927x over previous
import jax
import jax.numpy as jnp
from jax import lax
from jax.experimental import pallas as pl
from jax.experimental.pallas import tpu as pltpu
from jax.experimental.pallas import tpu_sc as plsc

B = 1024
READOUT_DIM = 16384
TOTAL = B * READOUT_DIM
NC, NS, L = 2, 16, 16
NW = NC * NS
FR, FI = 4, 8
NF = FR * FI
PLANE = TOTAL // FR
SN = 512
STEP_ELEMS = SN * FI
W_NODES = 16384
STEPS = W_NODES // SN
ROW = READOUT_DIM
PITCH = 17
GROUPS = SN // L
NBUF = 2


def _sc_body(f_hbm, w_hbm, o_hbm, fv0, fv1, wv0, wv1, ov0, ov1, mb,
             semf, semw, semo):
    fvs = (fv0, fv1)
    wvs = (wv0, wv1)
    ovs = (ov0, ov1)
    wid = lax.axis_index("s") * NC + lax.axis_index("c")
    in_base = wid * (W_NODES * FI)
    row_base = wid * STEPS

    def start_in(s, b):
        for fr in range(FR):
            a = fr * PLANE + in_base + s * STEP_ELEMS
            d = pl.ds(fr * STEP_ELEMS, STEP_ELEMS)
            pltpu.make_async_copy(f_hbm.at[pl.ds(a, STEP_ELEMS)], fvs[b].at[d], semf.at[b]).start()
            pltpu.make_async_copy(w_hbm.at[pl.ds(a, STEP_ELEMS)], wvs[b].at[d], semw.at[b]).start()

    def wait_in(s, b):
        for fr in range(FR):
            a = fr * PLANE + in_base + s * STEP_ELEMS
            d = pl.ds(fr * STEP_ELEMS, STEP_ELEMS)
            pltpu.make_async_copy(f_hbm.at[pl.ds(a, STEP_ELEMS)], fvs[b].at[d], semf.at[b]).wait()
            pltpu.make_async_copy(w_hbm.at[pl.ds(a, STEP_ELEMS)], wvs[b].at[d], semw.at[b]).wait()

    def start_out(s, b):
        g = row_base + s
        pltpu.make_async_copy(ovs[b], o_hbm.at[g >> 3, :, g & 7, :], semo.at[b]).start()

    def wait_out(s, b):
        g = row_base + s
        pltpu.make_async_copy(ovs[b], o_hbm.at[g >> 3, :, g & 7, :], semo.at[b]).wait()

    lanes = lax.iota(jnp.int32, L)
    lanes17 = lanes * PITCH

    start_in(0, 0)

    @pl.loop(0, STEPS, step=NBUF)
    def outer(s0):
        for b in range(NBUF):
            s = s0 + b

            @pl.when(s + 1 < STEPS)
            def _():
                start_in(s + 1, 1 - b)

            wait_in(s, b)

            @pl.when(s >= NBUF)
            def _():
                wait_out(s - NBUF, b)

            ob = ovs[b]
            fb, wb = fvs[b], wvs[b]

            @pl.loop(0, GROUPS)
            def _grp(g16):
                in_off = (g16 >> 3) * 1024 + (g16 & 7) * L

                @plsc.parallel_loop(0, NF, unroll=4, carry=lanes)
                def _p1(f, mrow):
                    off = (f >> 3) * STEP_ELEMS + (f & 7) * 128 + in_off
                    p = fb[pl.ds(off, L)] * wb[pl.ds(off, L)]
                    plsc.store_scatter(mb, [mrow], p)
                    return mrow + PITCH

                ov_base = g16 * (L * NF)

                @plsc.parallel_loop(0, L, unroll=4, carry=lanes17)
                def _p2(j, colA):
                    a = plsc.load_gather(mb, [colA])
                    c = plsc.load_gather(mb, [colA + L * PITCH])
                    o = ov_base + j * NF
                    row = o >> 7
                    col = o & 96
                    ob[row, pl.ds(col, L)] = a
                    ob[row, pl.ds(col + L, L)] = c
                    return colA + 1

            start_out(s, b)

    wait_out(STEPS - 2, 0)
    wait_out(STEPS - 1, 1)


def kernel(features, weights):
    def phys_flat(x):
        return x.reshape(4096, 128, FR, FI).transpose(2, 0, 3, 1).reshape(TOTAL)

    f2 = phys_flat(features)
    w2 = phys_flat(weights)
    mesh = plsc.VectorSubcoreMesh(
        core_axis_name="c", subcore_axis_name="s",
        num_cores=NC, num_subcores=NS)
    out = pl.kernel(
        _sc_body,
        out_type=jax.ShapeDtypeStruct((128, 128, 8, 128), jnp.float32),
        mesh=mesh,
        compiler_params=pltpu.CompilerParams(needs_layout_passes=False),
        scratch_types=[
            pltpu.VMEM((FR * STEP_ELEMS,), jnp.float32),
            pltpu.VMEM((FR * STEP_ELEMS,), jnp.float32),
            pltpu.VMEM((FR * STEP_ELEMS,), jnp.float32),
            pltpu.VMEM((FR * STEP_ELEMS,), jnp.float32),
            pltpu.VMEM((128, 128), jnp.float32),
            pltpu.VMEM((128, 128), jnp.float32),
            pltpu.VMEM((NF * PITCH,), jnp.float32),
            pltpu.SemaphoreType.DMA((NBUF,)),
            pltpu.SemaphoreType.DMA((NBUF,)),
            pltpu.SemaphoreType.DMA((NBUF,)),
        ],
    )(f2, w2)
    return out.transpose(0, 2, 1, 3).reshape(B, READOUT_DIM)
